# baseline (speedup 1.0000x reference)
import functools

import jax
import jax.numpy as jnp
import numpy as np
from jax import lax
from jax.experimental import pallas as pl
from jax.experimental.pallas import tpu as pltpu
from jax.experimental.pallas import tpu_sc as plsc

NC, NS, LANES = 2, 16, 16
NW = NC * NS


def _build_time_mats(wr, wi, T):
    tp = np.arange(T)[:, None].astype(np.float32)
    t = np.arange(T)[None, :].astype(np.float32)
    th = np.pi * (t - tp) / 2.0
    c = jnp.asarray(np.cos(th))
    s = jnp.asarray(np.sin(th))
    sign = jnp.asarray(((-1.0) ** (t + tp)).astype(np.float32))
    A = (wr[None, None, :, :, 0]
         + 2.0 * (c[:, :, None, None] * wr[None, None, :, :, 1]
                  - s[:, :, None, None] * wi[None, None, :, :, 1])
         + sign[:, :, None, None] * wr[None, None, :, :, 2]) / 4.0
    Ti, _, Cin, Cout = A.shape
    return jnp.transpose(A, (0, 2, 1, 3)).reshape(Ti * Cin, Ti * Cout)


def _prelude_body(h_ref, acat_ref, w1a_ref, w1b_ref, v_ref, bv_ref,
                  h2_ref, p_ref, q_ref, vn_ref):
    T = h_ref.shape[0]
    C = h_ref.shape[2]
    hcat = jnp.concatenate([h_ref[t] for t in range(T)], axis=-1)
    xh = jnp.dot(hcat, acat_ref[...], preferred_element_type=jnp.float32)
    h2 = hcat + jnp.where(xh > 0, xh, 0.2 * xh)
    h2_ref[...] = h2
    for t in range(T):
        blk = h2[:, t * C:(t + 1) * C]
        p_ref[t] = jnp.dot(blk, w1a_ref[...], preferred_element_type=jnp.float32)
        q_ref[t] = jnp.dot(blk, w1b_ref[...], preferred_element_type=jnp.float32)
    v = v_ref[...]
    vn_ref[...] = v + jnp.dot(v, bv_ref[...], preferred_element_type=jnp.float32)


def _prelude(h, acat, w1a, w1b, v, bv, T, BN, C, BR):
    nblk = BN // BR
    return pl.pallas_call(
        _prelude_body,
        grid=(nblk,),
        in_specs=[
            pl.BlockSpec((T, BR, C), lambda i: (0, i, 0)),
            pl.BlockSpec((T * C, T * C), lambda i: (0, 0)),
            pl.BlockSpec((C, C), lambda i: (0, 0)),
            pl.BlockSpec((C, C), lambda i: (0, 0)),
            pl.BlockSpec((BR, 3 * T), lambda i: (i, 0)),
            pl.BlockSpec((3 * T, 3 * T), lambda i: (0, 0)),
        ],
        out_specs=[
            pl.BlockSpec((BR, T * C), lambda i: (i, 0)),
            pl.BlockSpec((T, BR, C), lambda i: (0, i, 0)),
            pl.BlockSpec((T, BR, C), lambda i: (0, i, 0)),
            pl.BlockSpec((BR, 3 * T), lambda i: (i, 0)),
        ],
        out_shape=[
            jax.ShapeDtypeStruct((BN, T * C), jnp.float32),
            jax.ShapeDtypeStruct((T, BN, C), jnp.float32),
            jax.ShapeDtypeStruct((T, BN, C), jnp.float32),
            jax.ShapeDtypeStruct((BN, 3 * T), jnp.float32),
        ],
    )(h, acat, w1a, w1b, v, bv)


def _gather_kernel(TE, C, CE, tc_tiling=True, dtype=jnp.float32):
    per_w = TE // NW
    nch = per_w // CE

    @functools.partial(
        pl.kernel,
        out_type=[jax.ShapeDtypeStruct((TE, C), dtype),
                  jax.ShapeDtypeStruct((TE, C), dtype)],
        compiler_params=None if tc_tiling else pltpu.CompilerParams(
            use_tc_tiling_on_sc=False),
        mesh=plsc.VectorSubcoreMesh(core_axis_name="c", subcore_axis_name="s"),
        scratch_types=[
            pltpu.VMEM((CE,), jnp.int32),
            pltpu.VMEM((CE,), jnp.int32),
            pltpu.VMEM((CE, C), dtype),
            pltpu.VMEM((CE, C), dtype),
            pltpu.SemaphoreType.DMA,
            pltpu.SemaphoreType.DMA,
        ],
    )
    def k(ptab, qtab, src_hbm, dst_hbm, gp_hbm, gq_hbm,
          sidx, didx, gpb, gqb, sem1, sem2):
        wid = lax.axis_index("c") * NS + lax.axis_index("s")
        wbase = wid * per_w

        def body(ci, _):
            base = wbase + ci * CE
            pltpu.sync_copy(src_hbm.at[pl.ds(base, CE)], sidx)
            pltpu.sync_copy(dst_hbm.at[pl.ds(base, CE)], didx)
            cp1 = pltpu.async_copy(ptab.at[sidx], gpb, sem1)
            cp2 = pltpu.async_copy(qtab.at[didx], gqb, sem2)
            cp1.wait()
            cp2.wait()
            pltpu.sync_copy(gpb, gp_hbm.at[pl.ds(base, CE)])
            pltpu.sync_copy(gqb, gq_hbm.at[pl.ds(base, CE)])
            return 0

        lax.fori_loop(0, nch, body, 0)

    return k


def _gather_add_kernel(TE, C, CE, dtype=jnp.float32):
    per_w = TE // NW
    nch = per_w // CE
    niter = nch + 2
    ngrp = (niter + 2) // 3

    @functools.partial(
        pl.kernel,
        out_type=jax.ShapeDtypeStruct((TE, C), dtype),
        mesh=plsc.VectorSubcoreMesh(core_axis_name="c", subcore_axis_name="s"),
        scratch_types=[
            pltpu.VMEM((per_w,), jnp.int32),
            pltpu.VMEM((per_w,), jnp.int32),
            pltpu.VMEM((CE, C), dtype),
            pltpu.VMEM((CE, C), dtype),
            pltpu.VMEM((CE, C), dtype),
            pltpu.SemaphoreType.DMA,
            pltpu.SemaphoreType.DMA,
            pltpu.SemaphoreType.DMA,
            pltpu.SemaphoreType.DMA,
            pltpu.SemaphoreType.DMA,
            pltpu.SemaphoreType.DMA,
            pltpu.SemaphoreType.DMA,
            pltpu.SemaphoreType.DMA,
            pltpu.SemaphoreType.DMA,
        ],
    )
    def k(ptab, qtab, src_hbm, dst_hbm, g_hbm, sall, dall,
          gb0, gb1, gb2, qs0, qs1, qs2, ps0, ps1, ps2, ws0, ws1, ws2):
        wid = lax.axis_index("c") * NS + lax.axis_index("s")
        wbase = wid * per_w
        gbs = (gb0, gb1, gb2)
        qsems = (qs0, qs1, qs2)
        psems = (ps0, ps1, ps2)
        wsems = (ws0, ws1, ws2)
        pltpu.sync_copy(src_hbm.at[pl.ds(wbase, per_w)], sall)
        pltpu.sync_copy(dst_hbm.at[pl.ds(wbase, per_w)], dall)

        def group(g, _):
            for b3 in range(3):
                it = g * 3 + b3

                @pl.when(it < nch)
                def _():
                    b = b3
                    @pl.when(it >= 3)
                    def _():
                        pltpu.make_async_copy(
                            gbs[b], g_hbm.at[pl.ds(wbase + (it - 3) * CE, CE)],
                            wsems[b]).wait()
                    pltpu.async_copy(
                        qtab.at[dall.at[pl.ds(it * CE, CE)]], gbs[b], qsems[b])

                ci_p = it - 1
                @pl.when((ci_p >= 0) & (ci_p < nch))
                def _():
                    b = (b3 + 2) % 3
                    pltpu.make_async_copy(
                        qtab.at[dall.at[pl.ds(ci_p * CE, CE)]], gbs[b],
                        qsems[b]).wait()
                    pltpu.async_copy(
                        ptab.at[sall.at[pl.ds(ci_p * CE, CE)]], gbs[b],
                        psems[b], add=True)

                ci_w = it - 2
                @pl.when((ci_w >= 0) & (ci_w < nch))
                def _():
                    b = (b3 + 1) % 3
                    pltpu.make_async_copy(
                        ptab.at[sall.at[pl.ds(ci_w * CE, CE)]], gbs[b],
                        psems[b]).wait()
                    pltpu.async_copy(
                        gbs[b], g_hbm.at[pl.ds(wbase + ci_w * CE, CE)],
                        wsems[b])
            return 0

        lax.fori_loop(0, ngrp, group, 0)
        for k3 in range(3):
            ci = nch - 3 + k3
            if ci >= 0:
                b = ci % 3
                pltpu.make_async_copy(
                    gbs[b], g_hbm.at[pl.ds(wbase + ci * CE, CE)],
                    wsems[b]).wait()

    return k


def _edge_body(g_ref, xs_ref, xd_ref, w1c_ref, b1_ref, we2_ref, b2_ref,
               wg_ref, bg_ref, out_ref):
    rel = xs_ref[...] - xd_ref[...]
    mask3 = lax.broadcasted_iota(jnp.int32, (1, rel.shape[1]), 1) < 3
    d2 = jnp.sum(jnp.where(mask3, rel * rel, 0.0), axis=-1, keepdims=True)
    g = (g_ref[...] + d2 * w1c_ref[...] + b1_ref[...])
    m1 = g * jax.nn.sigmoid(g)
    m2 = jnp.dot(m1, we2_ref[...], preferred_element_type=jnp.float32) + b2_ref[...]
    m2 = m2 * jax.nn.sigmoid(m2)
    gate = jax.nn.sigmoid(
        jnp.sum(m2 * wg_ref[...] + bg_ref[...], axis=-1, keepdims=True))
    out_ref[...] = m2 * gate


def _edge_mlp_slice(t, g, xs, xd, w1c, b1, we2, b2, wgr, bgr, E, C, XW, BE):
    nblk = E // BE
    return pl.pallas_call(
        _edge_body,
        grid=(nblk,),
        in_specs=[
            pl.BlockSpec((BE, C), lambda i: (t * nblk + i, 0)),
            pl.BlockSpec((BE, XW), lambda i: (i, 0)),
            pl.BlockSpec((BE, XW), lambda i: (i, 0)),
            pl.BlockSpec((1, C), lambda i: (0, 0)),
            pl.BlockSpec((1, C), lambda i: (0, 0)),
            pl.BlockSpec((C, C), lambda i: (0, 0)),
            pl.BlockSpec((1, C), lambda i: (0, 0)),
            pl.BlockSpec((1, C), lambda i: (0, 0)),
            pl.BlockSpec((1, C), lambda i: (0, 0)),
        ],
        out_specs=pl.BlockSpec((BE, C), lambda i: (i, 0)),
        out_shape=jax.ShapeDtypeStruct((E, C), jnp.float32),
    )(g, xs, xd, w1c, b1, we2, b2, wgr, bgr)


def _scatter_slice_kernel(BN, E, C, CE):
    half = E // NC
    per_tile = half // NS
    nch = per_tile // CE
    niter = nch + 2
    ngrp = (niter + 3) // 4
    rows = (BN // NS) // 8 * 8
    tail = BN - rows * NS

    @functools.partial(
        pl.kernel,
        out_type=jax.ShapeDtypeStruct((NC * BN, C), jnp.float32),
        mesh=plsc.VectorSubcoreMesh(core_axis_name="c", subcore_axis_name="s"),
        scratch_types=[
            pltpu.VMEM_SHARED((BN, C), jnp.float32),
        ] + [pltpu.VMEM((CE, C), jnp.float32)] * 4
          + [pltpu.VMEM((CE,), jnp.int32)] * 4
          + [pltpu.SemaphoreType.DMA] * 12,
    )
    def k(m_hbm, dst_hbm, zero_hbm, agg_hbm, aggS,
          mb0, mb1, mb2, mb3, db0, db1, db2, db3,
          ms0, ms1, ms2, ms3, ds0, ds1, ds2, ds3, ss0, ss1, ss2, ss3):
        cid = lax.axis_index("c")
        sid = lax.axis_index("s")
        ebase0 = cid * half + sid * per_tile
        mbs = (mb0, mb1, mb2, mb3)
        dbs = (db0, db1, db2, db3)
        msems = (ms0, ms1, ms2, ms3)
        dsems = (ds0, ds1, ds2, ds3)
        ssems = (ss0, ss1, ss2, ss3)

        pltpu.sync_copy(zero_hbm.at[pl.ds(sid * rows, rows)],
                        aggS.at[pl.ds(sid * rows, rows)])
        if tail:
            @pl.when(sid == 0)
            def _():
                pltpu.sync_copy(zero_hbm.at[pl.ds(NS * rows, tail)],
                                aggS.at[pl.ds(NS * rows, tail)])
        plsc.subcore_barrier()

        def issue_loads(ci, b):
            pltpu.async_copy(m_hbm.at[pl.ds(ebase0 + ci * CE, CE)],
                             mbs[b], msems[b])
            pltpu.async_copy(dst_hbm.at[pl.ds(ebase0 + ci * CE, CE)],
                             dbs[b], dsems[b])

        def group(g, _):
            for b4 in range(4):
                it = g * 4 + b4

                @pl.when(it < nch)
                def _():
                    b = b4
                    @pl.when(it >= 4)
                    def _():
                        pltpu.make_async_copy(
                            mbs[b], aggS.at[dbs[b]], ssems[b]).wait()
                    issue_loads(it, b)

                ci = it - 2
                @pl.when((ci >= 0) & (ci < nch))
                def _():
                    b = (b4 + 2) % 4
                    pltpu.make_async_copy(
                        m_hbm.at[pl.ds(ebase0 + ci * CE, CE)],
                        mbs[b], msems[b]).wait()
                    pltpu.make_async_copy(
                        dst_hbm.at[pl.ds(ebase0 + ci * CE, CE)],
                        dbs[b], dsems[b]).wait()
                    pltpu.async_copy(mbs[b], aggS.at[dbs[b]], ssems[b],
                                     add=True)
            return 0

        lax.fori_loop(0, ngrp, group, 0)
        for k4 in range(4):
            ci = nch - 4 + k4
            if ci >= 0:
                b = ci % 4
                pltpu.make_async_copy(mbs[b], aggS.at[dbs[b]],
                                      ssems[b]).wait()
        plsc.subcore_barrier()
        pltpu.sync_copy(aggS.at[pl.ds(sid * rows, rows)],
                        agg_hbm.at[pl.ds(cid * BN + sid * rows, rows)])
        if tail:
            @pl.when(sid == 0)
            def _():
                pltpu.sync_copy(aggS.at[pl.ds(NS * rows, tail)],
                                agg_hbm.at[pl.ds(cid * BN + NS * rows, tail)])

    return k


def _node_body(h2_ref, a0_ref, a1_ref, wa_ref, wb_ref, b1_ref, w2_ref, b2_ref,
               out_ref):
    agg = a0_ref[...] + a1_ref[...]
    u = (jnp.dot(h2_ref[...], wa_ref[...], preferred_element_type=jnp.float32)
         + jnp.dot(agg, wb_ref[...], preferred_element_type=jnp.float32)
         + b1_ref[...])
    u = u * jax.nn.sigmoid(u)
    out_ref[...] = (h2_ref[...]
                    + jnp.dot(u, w2_ref[...], preferred_element_type=jnp.float32)
                    + b2_ref[...])


def _node_slice(t, h2cat, part, wa, wb, b1, w2, b2, BN, C, BR):
    nblk = BN // BR
    return pl.pallas_call(
        _node_body,
        grid=(nblk,),
        in_specs=[
            pl.BlockSpec((BR, C), lambda i: (i, t)),
            pl.BlockSpec((BR, C), lambda i: (i, 0)),
            pl.BlockSpec((BR, C), lambda i: (nblk + i, 0)),
            pl.BlockSpec((C, C), lambda i: (0, 0)),
            pl.BlockSpec((C, C), lambda i: (0, 0)),
            pl.BlockSpec((1, C), lambda i: (0, 0)),
            pl.BlockSpec((C, C), lambda i: (0, 0)),
            pl.BlockSpec((1, C), lambda i: (0, 0)),
        ],
        out_specs=pl.BlockSpec((BR, C), lambda i: (i, 0)),
        out_shape=jax.ShapeDtypeStruct((BN, C), jnp.float32),
    )(h2cat, part, part, wa, wb, b1, w2, b2)


def kernel(h, x, vel_all, edge_index, tc_h_wr, tc_h_wi, tc_v_wr, tc_v_wi,
           W_e1, b_e1, W_e2, b_e2, W_g, b_g, W_n1, b_n1, W_n2, b_n2):
    T, BN, C = h.shape
    E = edge_index.shape[1]
    TE = T * E

    acat = _build_time_mats(tc_h_wr, tc_h_wi, T)
    a_v = _build_time_mats(tc_v_wr, tc_v_wi, T)
    bv = jnp.kron(a_v, jnp.eye(3, dtype=jnp.float32))
    w1a, w1b = W_e1[:C], W_e1[C:2 * C]
    w1c = W_e1[2 * C].reshape(1, C)
    src0 = edge_index[0].astype(jnp.int32)
    dst0 = edge_index[1].astype(jnp.int32)

    vflat = vel_all.reshape(BN, T * 3)
    h2cat, ptab, qtab, vnew = _prelude(h, acat, w1a, w1b, vflat, bv,
                                       T, BN, C, BR=2000)
    vel_out = vnew.reshape(BN, T, 3)

    XW = 16
    x16 = jnp.zeros((BN, XW), jnp.float32).at[:, :3].set(x)
    xs_g, xd_g = _gather_kernel(E, XW, CE=1000, tc_tiling=False)(
        x16, x16, src0, dst0)

    offs = jnp.repeat(jnp.arange(T, dtype=jnp.int32) * BN, E)
    src_all = jnp.tile(src0, T) + offs
    dst_all = jnp.tile(dst0, T) + offs
    g = _gather_add_kernel(TE, C, CE=80)(
        ptab.reshape(T * BN, C), qtab.reshape(T * BN, C), src_all, dst_all)

    bgr = jnp.full((1, C), b_g[0] / C, jnp.float32)
    zeros = jnp.zeros((BN, C), jnp.float32)
    scat = _scatter_slice_kernel(BN, E, C, CE=40)
    h_outs = []
    for t in range(T):
        m_t = _edge_mlp_slice(t, g, xs_g, xd_g, w1c, b_e1.reshape(1, C), W_e2,
                              b_e2.reshape(1, C), W_g.reshape(1, C), bgr,
                              E, C, XW, BE=2000)
        part_t = scat(m_t, dst0, zeros)
        h_outs.append(_node_slice(t, h2cat, part_t, W_n1[:C], W_n1[C:],
                                  b_n1.reshape(1, C), W_n2,
                                  b_n2.reshape(1, C), BN, C, BR=2000))
    return jnp.stack(h_outs, axis=0), vel_out


# speedup vs baseline: 1.0027x; 1.0027x over previous
import functools

import jax
import jax.numpy as jnp
import numpy as np
from jax import lax
from jax.experimental import pallas as pl
from jax.experimental.pallas import tpu as pltpu
from jax.experimental.pallas import tpu_sc as plsc

NC, NS, LANES = 2, 16, 16
NW = NC * NS


def _build_time_mats(wr, wi, T):
    tp = np.arange(T)[:, None].astype(np.float32)
    t = np.arange(T)[None, :].astype(np.float32)
    th = np.pi * (t - tp) / 2.0
    c = jnp.asarray(np.cos(th))
    s = jnp.asarray(np.sin(th))
    sign = jnp.asarray(((-1.0) ** (t + tp)).astype(np.float32))
    A = (wr[None, None, :, :, 0]
         + 2.0 * (c[:, :, None, None] * wr[None, None, :, :, 1]
                  - s[:, :, None, None] * wi[None, None, :, :, 1])
         + sign[:, :, None, None] * wr[None, None, :, :, 2]) / 4.0
    Ti, _, Cin, Cout = A.shape
    return jnp.transpose(A, (0, 2, 1, 3)).reshape(Ti * Cin, Ti * Cout)


def _prelude_body(h_ref, acat_ref, w1a_ref, w1b_ref, v_ref, bv_ref,
                  h2_ref, p_ref, q_ref, vn_ref):
    T = h_ref.shape[0]
    C = h_ref.shape[2]
    hcat = jnp.concatenate([h_ref[t] for t in range(T)], axis=-1)
    xh = jnp.dot(hcat, acat_ref[...], preferred_element_type=jnp.float32)
    h2 = hcat + jnp.where(xh > 0, xh, 0.2 * xh)
    h2_ref[...] = h2
    for t in range(T):
        blk = h2[:, t * C:(t + 1) * C]
        p_ref[t] = jnp.dot(blk, w1a_ref[...], preferred_element_type=jnp.float32)
        q_ref[t] = jnp.dot(blk, w1b_ref[...], preferred_element_type=jnp.float32)
    v = v_ref[...]
    vn_ref[...] = v + jnp.dot(v, bv_ref[...], preferred_element_type=jnp.float32)


def _prelude(h, acat, w1a, w1b, v, bv, T, BN, C, BR):
    nblk = BN // BR
    return pl.pallas_call(
        _prelude_body,
        grid=(nblk,),
        in_specs=[
            pl.BlockSpec((T, BR, C), lambda i: (0, i, 0)),
            pl.BlockSpec((T * C, T * C), lambda i: (0, 0)),
            pl.BlockSpec((C, C), lambda i: (0, 0)),
            pl.BlockSpec((C, C), lambda i: (0, 0)),
            pl.BlockSpec((BR, 3 * T), lambda i: (i, 0)),
            pl.BlockSpec((3 * T, 3 * T), lambda i: (0, 0)),
        ],
        out_specs=[
            pl.BlockSpec((BR, T * C), lambda i: (i, 0)),
            pl.BlockSpec((T, BR, C), lambda i: (0, i, 0)),
            pl.BlockSpec((T, BR, C), lambda i: (0, i, 0)),
            pl.BlockSpec((BR, 3 * T), lambda i: (i, 0)),
        ],
        out_shape=[
            jax.ShapeDtypeStruct((BN, T * C), jnp.float32),
            jax.ShapeDtypeStruct((T, BN, C), jnp.float32),
            jax.ShapeDtypeStruct((T, BN, C), jnp.float32),
            jax.ShapeDtypeStruct((BN, 3 * T), jnp.float32),
        ],
    )(h, acat, w1a, w1b, v, bv)


def _gather_kernel(TE, C, CE, tc_tiling=True, dtype=jnp.float32):
    per_w = TE // NW
    nch = per_w // CE

    @functools.partial(
        pl.kernel,
        out_type=[jax.ShapeDtypeStruct((TE, C), dtype),
                  jax.ShapeDtypeStruct((TE, C), dtype)],
        compiler_params=None if tc_tiling else pltpu.CompilerParams(
            use_tc_tiling_on_sc=False),
        mesh=plsc.VectorSubcoreMesh(core_axis_name="c", subcore_axis_name="s"),
        scratch_types=[
            pltpu.VMEM((CE,), jnp.int32),
            pltpu.VMEM((CE,), jnp.int32),
            pltpu.VMEM((CE, C), dtype),
            pltpu.VMEM((CE, C), dtype),
            pltpu.SemaphoreType.DMA,
            pltpu.SemaphoreType.DMA,
        ],
    )
    def k(ptab, qtab, src_hbm, dst_hbm, gp_hbm, gq_hbm,
          sidx, didx, gpb, gqb, sem1, sem2):
        wid = lax.axis_index("c") * NS + lax.axis_index("s")
        wbase = wid * per_w

        def body(ci, _):
            base = wbase + ci * CE
            pltpu.sync_copy(src_hbm.at[pl.ds(base, CE)], sidx)
            pltpu.sync_copy(dst_hbm.at[pl.ds(base, CE)], didx)
            cp1 = pltpu.async_copy(ptab.at[sidx], gpb, sem1)
            cp2 = pltpu.async_copy(qtab.at[didx], gqb, sem2)
            cp1.wait()
            cp2.wait()
            pltpu.sync_copy(gpb, gp_hbm.at[pl.ds(base, CE)])
            pltpu.sync_copy(gqb, gq_hbm.at[pl.ds(base, CE)])
            return 0

        lax.fori_loop(0, nch, body, 0)

    return k


def _gather_add_kernel(TE, C, CE, dtype=jnp.float32):
    per_w = TE // NW
    nch = per_w // CE
    niter = nch + 2
    ngrp = (niter + 2) // 3

    @functools.partial(
        pl.kernel,
        out_type=jax.ShapeDtypeStruct((TE, C), dtype),
        mesh=plsc.VectorSubcoreMesh(core_axis_name="c", subcore_axis_name="s"),
        scratch_types=[
            pltpu.VMEM((per_w,), jnp.int32),
            pltpu.VMEM((per_w,), jnp.int32),
            pltpu.VMEM((CE, C), dtype),
            pltpu.VMEM((CE, C), dtype),
            pltpu.VMEM((CE, C), dtype),
            pltpu.SemaphoreType.DMA,
            pltpu.SemaphoreType.DMA,
            pltpu.SemaphoreType.DMA,
            pltpu.SemaphoreType.DMA,
            pltpu.SemaphoreType.DMA,
            pltpu.SemaphoreType.DMA,
            pltpu.SemaphoreType.DMA,
            pltpu.SemaphoreType.DMA,
            pltpu.SemaphoreType.DMA,
        ],
    )
    def k(ptab, qtab, src_hbm, dst_hbm, g_hbm, sall, dall,
          gb0, gb1, gb2, qs0, qs1, qs2, ps0, ps1, ps2, ws0, ws1, ws2):
        wid = lax.axis_index("c") * NS + lax.axis_index("s")
        wbase = wid * per_w
        gbs = (gb0, gb1, gb2)
        qsems = (qs0, qs1, qs2)
        psems = (ps0, ps1, ps2)
        wsems = (ws0, ws1, ws2)
        pltpu.sync_copy(src_hbm.at[pl.ds(wbase, per_w)], sall)
        pltpu.sync_copy(dst_hbm.at[pl.ds(wbase, per_w)], dall)

        def group(g, _):
            for b3 in range(3):
                it = g * 3 + b3

                @pl.when(it < nch)
                def _():
                    b = b3
                    @pl.when(it >= 3)
                    def _():
                        pltpu.make_async_copy(
                            gbs[b], g_hbm.at[pl.ds(wbase + (it - 3) * CE, CE)],
                            wsems[b]).wait()
                    pltpu.async_copy(
                        qtab.at[dall.at[pl.ds(it * CE, CE)]], gbs[b], qsems[b])

                ci_p = it - 1
                @pl.when((ci_p >= 0) & (ci_p < nch))
                def _():
                    b = (b3 + 2) % 3
                    pltpu.make_async_copy(
                        qtab.at[dall.at[pl.ds(ci_p * CE, CE)]], gbs[b],
                        qsems[b]).wait()
                    pltpu.async_copy(
                        ptab.at[sall.at[pl.ds(ci_p * CE, CE)]], gbs[b],
                        psems[b], add=True)

                ci_w = it - 2
                @pl.when((ci_w >= 0) & (ci_w < nch))
                def _():
                    b = (b3 + 1) % 3
                    pltpu.make_async_copy(
                        ptab.at[sall.at[pl.ds(ci_w * CE, CE)]], gbs[b],
                        psems[b]).wait()
                    pltpu.async_copy(
                        gbs[b], g_hbm.at[pl.ds(wbase + ci_w * CE, CE)],
                        wsems[b])
            return 0

        lax.fori_loop(0, ngrp, group, 0)
        for k3 in range(3):
            ci = nch - 3 + k3
            if ci >= 0:
                b = ci % 3
                pltpu.make_async_copy(
                    gbs[b], g_hbm.at[pl.ds(wbase + ci * CE, CE)],
                    wsems[b]).wait()

    return k


def _edge_body(g_ref, xs_ref, xd_ref, w1c_ref, b1_ref, we2_ref, b2_ref,
               wg_ref, bg_ref, out_ref):
    rel = xs_ref[...] - xd_ref[...]
    mask3 = lax.broadcasted_iota(jnp.int32, (1, rel.shape[1]), 1) < 3
    d2 = jnp.sum(jnp.where(mask3, rel * rel, 0.0), axis=-1, keepdims=True)
    g = (g_ref[...] + d2 * w1c_ref[...] + b1_ref[...])
    m1 = g * jax.nn.sigmoid(g)
    m2 = jnp.dot(m1, we2_ref[...], preferred_element_type=jnp.float32) + b2_ref[...]
    m2 = m2 * jax.nn.sigmoid(m2)
    gate = jax.nn.sigmoid(
        jnp.sum(m2 * wg_ref[...] + bg_ref[...], axis=-1, keepdims=True))
    out_ref[...] = m2 * gate


def _edge_mlp_slice(t, g, xs, xd, w1c, b1, we2, b2, wgr, bgr, E, C, XW, BE):
    nblk = E // BE
    return pl.pallas_call(
        _edge_body,
        grid=(nblk,),
        in_specs=[
            pl.BlockSpec((BE, C), lambda i: (t * nblk + i, 0)),
            pl.BlockSpec((BE, XW), lambda i: (i, 0)),
            pl.BlockSpec((BE, XW), lambda i: (i, 0)),
            pl.BlockSpec((1, C), lambda i: (0, 0)),
            pl.BlockSpec((1, C), lambda i: (0, 0)),
            pl.BlockSpec((C, C), lambda i: (0, 0)),
            pl.BlockSpec((1, C), lambda i: (0, 0)),
            pl.BlockSpec((1, C), lambda i: (0, 0)),
            pl.BlockSpec((1, C), lambda i: (0, 0)),
        ],
        out_specs=pl.BlockSpec((BE, C), lambda i: (i, 0)),
        out_shape=jax.ShapeDtypeStruct((E, C), jnp.float32),
    )(g, xs, xd, w1c, b1, we2, b2, wgr, bgr)


def _scatter_slice_kernel(BN, E, C, CE):
    half = E // NC
    per_tile = half // NS
    nch = per_tile // CE
    niter = nch + 2
    ngrp = (niter + 3) // 4
    rows = (BN // NS) // 8 * 8
    tail = BN - rows * NS

    @functools.partial(
        pl.kernel,
        out_type=jax.ShapeDtypeStruct((NC * BN, C), jnp.float32),
        mesh=plsc.VectorSubcoreMesh(core_axis_name="c", subcore_axis_name="s"),
        scratch_types=[
            pltpu.VMEM_SHARED((BN, C), jnp.float32),
        ] + [pltpu.VMEM((CE, C), jnp.float32)] * 4
          + [pltpu.VMEM((CE,), jnp.int32)] * 4
          + [pltpu.SemaphoreType.DMA] * 12,
    )
    def k(m_hbm, dst_hbm, zero_hbm, agg_hbm, aggS,
          mb0, mb1, mb2, mb3, db0, db1, db2, db3,
          ms0, ms1, ms2, ms3, ds0, ds1, ds2, ds3, ss0, ss1, ss2, ss3):
        cid = lax.axis_index("c")
        sid = lax.axis_index("s")
        ebase0 = cid * half + sid * per_tile
        mbs = (mb0, mb1, mb2, mb3)
        dbs = (db0, db1, db2, db3)
        msems = (ms0, ms1, ms2, ms3)
        dsems = (ds0, ds1, ds2, ds3)
        ssems = (ss0, ss1, ss2, ss3)

        pltpu.sync_copy(zero_hbm.at[pl.ds(sid * rows, rows)],
                        aggS.at[pl.ds(sid * rows, rows)])
        if tail:
            @pl.when(sid == 0)
            def _():
                pltpu.sync_copy(zero_hbm.at[pl.ds(NS * rows, tail)],
                                aggS.at[pl.ds(NS * rows, tail)])
        plsc.subcore_barrier()

        def issue_loads(ci, b):
            pltpu.async_copy(m_hbm.at[pl.ds(ebase0 + ci * CE, CE)],
                             mbs[b], msems[b])
            pltpu.async_copy(dst_hbm.at[pl.ds(ebase0 + ci * CE, CE)],
                             dbs[b], dsems[b])

        def group(g, _):
            for b4 in range(4):
                it = g * 4 + b4

                @pl.when(it < nch)
                def _():
                    b = b4
                    @pl.when(it >= 4)
                    def _():
                        pltpu.make_async_copy(
                            mbs[b], aggS.at[dbs[b]], ssems[b]).wait()
                    issue_loads(it, b)

                ci = it - 2
                @pl.when((ci >= 0) & (ci < nch))
                def _():
                    b = (b4 + 2) % 4
                    pltpu.make_async_copy(
                        m_hbm.at[pl.ds(ebase0 + ci * CE, CE)],
                        mbs[b], msems[b]).wait()
                    pltpu.make_async_copy(
                        dst_hbm.at[pl.ds(ebase0 + ci * CE, CE)],
                        dbs[b], dsems[b]).wait()
                    pltpu.async_copy(mbs[b], aggS.at[dbs[b]], ssems[b],
                                     add=True)
            return 0

        lax.fori_loop(0, ngrp, group, 0)
        for k4 in range(4):
            ci = nch - 4 + k4
            if ci >= 0:
                b = ci % 4
                pltpu.make_async_copy(mbs[b], aggS.at[dbs[b]],
                                      ssems[b]).wait()
        plsc.subcore_barrier()
        pltpu.sync_copy(aggS.at[pl.ds(sid * rows, rows)],
                        agg_hbm.at[pl.ds(cid * BN + sid * rows, rows)])
        if tail:
            @pl.when(sid == 0)
            def _():
                pltpu.sync_copy(aggS.at[pl.ds(NS * rows, tail)],
                                agg_hbm.at[pl.ds(cid * BN + NS * rows, tail)])

    return k


def _node_body(h2_ref, a0_ref, a1_ref, wa_ref, wb_ref, b1_ref, w2_ref, b2_ref,
               out_ref):
    agg = a0_ref[...] + a1_ref[...]
    u = (jnp.dot(h2_ref[...], wa_ref[...], preferred_element_type=jnp.float32)
         + jnp.dot(agg, wb_ref[...], preferred_element_type=jnp.float32)
         + b1_ref[...])
    u = u * jax.nn.sigmoid(u)
    out_ref[...] = (h2_ref[...]
                    + jnp.dot(u, w2_ref[...], preferred_element_type=jnp.float32)
                    + b2_ref[...])


def _node_slice(t, h2cat, part, wa, wb, b1, w2, b2, BN, C, BR):
    nblk = BN // BR
    return pl.pallas_call(
        _node_body,
        grid=(nblk,),
        in_specs=[
            pl.BlockSpec((BR, C), lambda i: (i, t)),
            pl.BlockSpec((BR, C), lambda i: (i, 0)),
            pl.BlockSpec((BR, C), lambda i: (nblk + i, 0)),
            pl.BlockSpec((C, C), lambda i: (0, 0)),
            pl.BlockSpec((C, C), lambda i: (0, 0)),
            pl.BlockSpec((1, C), lambda i: (0, 0)),
            pl.BlockSpec((C, C), lambda i: (0, 0)),
            pl.BlockSpec((1, C), lambda i: (0, 0)),
        ],
        out_specs=pl.BlockSpec((BR, C), lambda i: (i, 0)),
        out_shape=jax.ShapeDtypeStruct((BN, C), jnp.float32),
    )(h2cat, part, part, wa, wb, b1, w2, b2)


def kernel(h, x, vel_all, edge_index, tc_h_wr, tc_h_wi, tc_v_wr, tc_v_wi,
           W_e1, b_e1, W_e2, b_e2, W_g, b_g, W_n1, b_n1, W_n2, b_n2):
    T, BN, C = h.shape
    E = edge_index.shape[1]
    TE = T * E

    acat = _build_time_mats(tc_h_wr, tc_h_wi, T)
    a_v = _build_time_mats(tc_v_wr, tc_v_wi, T)
    bv = jnp.kron(a_v, jnp.eye(3, dtype=jnp.float32))
    w1a, w1b = W_e1[:C], W_e1[C:2 * C]
    w1c = W_e1[2 * C].reshape(1, C)
    src0 = edge_index[0].astype(jnp.int32)
    dst0 = edge_index[1].astype(jnp.int32)

    vflat = vel_all.reshape(BN, T * 3)
    h2cat, ptab, qtab, vnew = _prelude(h, acat, w1a, w1b, vflat, bv,
                                       T, BN, C, BR=2000)
    vel_out = vnew.reshape(BN, T, 3)

    XW = 16
    x16 = jnp.zeros((BN, XW), jnp.float32).at[:, :3].set(x)
    xs_g, xd_g = _gather_kernel(E, XW, CE=1000, tc_tiling=False)(
        x16, x16, src0, dst0)

    offs = jnp.repeat(jnp.arange(T, dtype=jnp.int32) * BN, E)
    src_all = jnp.tile(src0, T) + offs
    dst_all = jnp.tile(dst0, T) + offs
    g = _gather_add_kernel(TE, C, CE=200)(
        ptab.reshape(T * BN, C), qtab.reshape(T * BN, C), src_all, dst_all)

    bgr = jnp.full((1, C), b_g[0] / C, jnp.float32)
    zeros = jnp.zeros((BN, C), jnp.float32)
    scat = _scatter_slice_kernel(BN, E, C, CE=40)
    h_outs = []
    for t in range(T):
        m_t = _edge_mlp_slice(t, g, xs_g, xd_g, w1c, b_e1.reshape(1, C), W_e2,
                              b_e2.reshape(1, C), W_g.reshape(1, C), bgr,
                              E, C, XW, BE=2000)
        part_t = scat(m_t, dst0, zeros)
        h_outs.append(_node_slice(t, h2cat, part_t, W_n1[:C], W_n1[C:],
                                  b_n1.reshape(1, C), W_n2,
                                  b_n2.reshape(1, C), BN, C, BR=2000))
    return jnp.stack(h_outs, axis=0), vel_out


# speedup vs baseline: 1.0621x; 1.0593x over previous
import functools

import jax
import jax.numpy as jnp
import numpy as np
from jax import lax
from jax.experimental import pallas as pl
from jax.experimental.pallas import tpu as pltpu
from jax.experimental.pallas import tpu_sc as plsc

NC, NS, LANES = 2, 16, 16
NW = NC * NS


def _build_time_mats(wr, wi, T):
    tp = np.arange(T)[:, None].astype(np.float32)
    t = np.arange(T)[None, :].astype(np.float32)
    th = np.pi * (t - tp) / 2.0
    c = jnp.asarray(np.cos(th))
    s = jnp.asarray(np.sin(th))
    sign = jnp.asarray(((-1.0) ** (t + tp)).astype(np.float32))
    A = (wr[None, None, :, :, 0]
         + 2.0 * (c[:, :, None, None] * wr[None, None, :, :, 1]
                  - s[:, :, None, None] * wi[None, None, :, :, 1])
         + sign[:, :, None, None] * wr[None, None, :, :, 2]) / 4.0
    Ti, _, Cin, Cout = A.shape
    return jnp.transpose(A, (0, 2, 1, 3)).reshape(Ti * Cin, Ti * Cout)


def _prelude_body(h_ref, acat_ref, w1a_ref, w1b_ref, v_ref, bv_ref,
                  h2_ref, p_ref, q_ref, vn_ref):
    T = h_ref.shape[0]
    C = h_ref.shape[2]
    hcat = jnp.concatenate([h_ref[t] for t in range(T)], axis=-1)
    xh = jnp.dot(hcat, acat_ref[...], preferred_element_type=jnp.float32)
    h2 = hcat + jnp.where(xh > 0, xh, 0.2 * xh)
    h2_ref[...] = h2
    for t in range(T):
        blk = h2[:, t * C:(t + 1) * C]
        p_ref[t] = jnp.dot(blk, w1a_ref[...], preferred_element_type=jnp.float32)
        q_ref[t] = jnp.dot(blk, w1b_ref[...], preferred_element_type=jnp.float32)
    v = v_ref[...]
    vn_ref[...] = v + jnp.dot(v, bv_ref[...], preferred_element_type=jnp.float32)


def _prelude(h, acat, w1a, w1b, v, bv, T, BN, C, BR):
    nblk = BN // BR
    return pl.pallas_call(
        _prelude_body,
        grid=(nblk,),
        in_specs=[
            pl.BlockSpec((T, BR, C), lambda i: (0, i, 0)),
            pl.BlockSpec((T * C, T * C), lambda i: (0, 0)),
            pl.BlockSpec((C, C), lambda i: (0, 0)),
            pl.BlockSpec((C, C), lambda i: (0, 0)),
            pl.BlockSpec((BR, 3 * T), lambda i: (i, 0)),
            pl.BlockSpec((3 * T, 3 * T), lambda i: (0, 0)),
        ],
        out_specs=[
            pl.BlockSpec((BR, T * C), lambda i: (i, 0)),
            pl.BlockSpec((T, BR, C), lambda i: (0, i, 0)),
            pl.BlockSpec((T, BR, C), lambda i: (0, i, 0)),
            pl.BlockSpec((BR, 3 * T), lambda i: (i, 0)),
        ],
        out_shape=[
            jax.ShapeDtypeStruct((BN, T * C), jnp.float32),
            jax.ShapeDtypeStruct((T, BN, C), jnp.float32),
            jax.ShapeDtypeStruct((T, BN, C), jnp.float32),
            jax.ShapeDtypeStruct((BN, 3 * T), jnp.float32),
        ],
    )(h, acat, w1a, w1b, v, bv)


def _gather_kernel(TE, C, CE, tc_tiling=True, dtype=jnp.float32):
    per_w = TE // NW
    nch = per_w // CE

    @functools.partial(
        pl.kernel,
        out_type=[jax.ShapeDtypeStruct((TE, C), dtype),
                  jax.ShapeDtypeStruct((TE, C), dtype)],
        compiler_params=None if tc_tiling else pltpu.CompilerParams(
            use_tc_tiling_on_sc=False),
        mesh=plsc.VectorSubcoreMesh(core_axis_name="c", subcore_axis_name="s"),
        scratch_types=[
            pltpu.VMEM((CE,), jnp.int32),
            pltpu.VMEM((CE,), jnp.int32),
            pltpu.VMEM((CE, C), dtype),
            pltpu.VMEM((CE, C), dtype),
            pltpu.SemaphoreType.DMA,
            pltpu.SemaphoreType.DMA,
        ],
    )
    def k(ptab, qtab, src_hbm, dst_hbm, gp_hbm, gq_hbm,
          sidx, didx, gpb, gqb, sem1, sem2):
        wid = lax.axis_index("c") * NS + lax.axis_index("s")
        wbase = wid * per_w

        def body(ci, _):
            base = wbase + ci * CE
            pltpu.sync_copy(src_hbm.at[pl.ds(base, CE)], sidx)
            pltpu.sync_copy(dst_hbm.at[pl.ds(base, CE)], didx)
            cp1 = pltpu.async_copy(ptab.at[sidx], gpb, sem1)
            cp2 = pltpu.async_copy(qtab.at[didx], gqb, sem2)
            cp1.wait()
            cp2.wait()
            pltpu.sync_copy(gpb, gp_hbm.at[pl.ds(base, CE)])
            pltpu.sync_copy(gqb, gq_hbm.at[pl.ds(base, CE)])
            return 0

        lax.fori_loop(0, nch, body, 0)

    return k


def _gather_add_kernel(TE, C, CE, dtype=jnp.float32):
    per_w = TE // NW
    nch = per_w // CE
    niter = nch + 2
    ngrp = (niter + 2) // 3

    @functools.partial(
        pl.kernel,
        out_type=jax.ShapeDtypeStruct((TE, C), dtype),
        mesh=plsc.VectorSubcoreMesh(core_axis_name="c", subcore_axis_name="s"),
        scratch_types=[
            pltpu.VMEM((per_w,), jnp.int32),
            pltpu.VMEM((per_w,), jnp.int32),
            pltpu.VMEM((CE, C), dtype),
            pltpu.VMEM((CE, C), dtype),
            pltpu.VMEM((CE, C), dtype),
            pltpu.SemaphoreType.DMA,
            pltpu.SemaphoreType.DMA,
            pltpu.SemaphoreType.DMA,
            pltpu.SemaphoreType.DMA,
            pltpu.SemaphoreType.DMA,
            pltpu.SemaphoreType.DMA,
            pltpu.SemaphoreType.DMA,
            pltpu.SemaphoreType.DMA,
            pltpu.SemaphoreType.DMA,
        ],
    )
    def k(ptab, qtab, src_hbm, dst_hbm, g_hbm, sall, dall,
          gb0, gb1, gb2, qs0, qs1, qs2, ps0, ps1, ps2, ws0, ws1, ws2):
        wid = lax.axis_index("c") * NS + lax.axis_index("s")
        wbase = wid * per_w
        gbs = (gb0, gb1, gb2)
        qsems = (qs0, qs1, qs2)
        psems = (ps0, ps1, ps2)
        wsems = (ws0, ws1, ws2)
        pltpu.sync_copy(src_hbm.at[pl.ds(wbase, per_w)], sall)
        pltpu.sync_copy(dst_hbm.at[pl.ds(wbase, per_w)], dall)

        def group(g, _):
            for b3 in range(3):
                it = g * 3 + b3

                @pl.when(it < nch)
                def _():
                    b = b3
                    @pl.when(it >= 3)
                    def _():
                        pltpu.make_async_copy(
                            gbs[b], g_hbm.at[pl.ds(wbase + (it - 3) * CE, CE)],
                            wsems[b]).wait()
                    pltpu.async_copy(
                        qtab.at[dall.at[pl.ds(it * CE, CE)]], gbs[b], qsems[b])

                ci_p = it - 1
                @pl.when((ci_p >= 0) & (ci_p < nch))
                def _():
                    b = (b3 + 2) % 3
                    pltpu.make_async_copy(
                        qtab.at[dall.at[pl.ds(ci_p * CE, CE)]], gbs[b],
                        qsems[b]).wait()
                    pltpu.async_copy(
                        ptab.at[sall.at[pl.ds(ci_p * CE, CE)]], gbs[b],
                        psems[b], add=True)

                ci_w = it - 2
                @pl.when((ci_w >= 0) & (ci_w < nch))
                def _():
                    b = (b3 + 1) % 3
                    pltpu.make_async_copy(
                        ptab.at[sall.at[pl.ds(ci_w * CE, CE)]], gbs[b],
                        psems[b]).wait()
                    pltpu.async_copy(
                        gbs[b], g_hbm.at[pl.ds(wbase + ci_w * CE, CE)],
                        wsems[b])
            return 0

        lax.fori_loop(0, ngrp, group, 0)
        for k3 in range(3):
            ci = nch - 3 + k3
            if ci >= 0:
                b = ci % 3
                pltpu.make_async_copy(
                    gbs[b], g_hbm.at[pl.ds(wbase + ci * CE, CE)],
                    wsems[b]).wait()

    return k


def _edge_body(g_ref, xs_ref, xd_ref, w1c_ref, b1_ref, we2_ref, b2_ref,
               wg_ref, bg_ref, out_ref):
    rel = xs_ref[...] - xd_ref[...]
    mask3 = lax.broadcasted_iota(jnp.int32, (1, rel.shape[1]), 1) < 3
    d2 = jnp.sum(jnp.where(mask3, rel * rel, 0.0), axis=-1, keepdims=True)
    g = (g_ref[...] + d2 * w1c_ref[...] + b1_ref[...])
    m1 = g * jax.nn.sigmoid(g)
    m2 = jnp.dot(m1, we2_ref[...], preferred_element_type=jnp.float32) + b2_ref[...]
    m2 = m2 * jax.nn.sigmoid(m2)
    gate = jax.nn.sigmoid(
        jnp.sum(m2 * wg_ref[...] + bg_ref[...], axis=-1, keepdims=True))
    out_ref[...] = m2 * gate


def _edge_mlp_slice(t, g, xs, xd, w1c, b1, we2, b2, wgr, bgr, E, C, XW, BE):
    nblk = E // BE
    return pl.pallas_call(
        _edge_body,
        grid=(nblk,),
        in_specs=[
            pl.BlockSpec((BE, C), lambda i: (t * nblk + i, 0)),
            pl.BlockSpec((BE, XW), lambda i: (i, 0)),
            pl.BlockSpec((BE, XW), lambda i: (i, 0)),
            pl.BlockSpec((1, C), lambda i: (0, 0)),
            pl.BlockSpec((1, C), lambda i: (0, 0)),
            pl.BlockSpec((C, C), lambda i: (0, 0)),
            pl.BlockSpec((1, C), lambda i: (0, 0)),
            pl.BlockSpec((1, C), lambda i: (0, 0)),
            pl.BlockSpec((1, C), lambda i: (0, 0)),
        ],
        out_specs=pl.BlockSpec((BE, C), lambda i: (i, 0)),
        out_shape=jax.ShapeDtypeStruct((E, C), jnp.float32),
    )(g, xs, xd, w1c, b1, we2, b2, wgr, bgr)


def _scatter_slice_kernel(BN, E, C, CE):
    half = E // NC
    per_tile = half // NS
    nch = per_tile // CE
    niter = nch + 2
    ngrp = (niter + 3) // 4
    rows = (BN // NS) // 8 * 8
    tail = BN - rows * NS

    @functools.partial(
        pl.kernel,
        out_type=jax.ShapeDtypeStruct((NC * BN, C), jnp.float32),
        mesh=plsc.VectorSubcoreMesh(core_axis_name="c", subcore_axis_name="s"),
        scratch_types=[
            pltpu.VMEM_SHARED((BN, C), jnp.float32),
        ] + [pltpu.VMEM((CE, C), jnp.float32)] * 4
          + [pltpu.VMEM((CE,), jnp.int32)] * 4
          + [pltpu.SemaphoreType.DMA] * 12,
    )
    def k(m_hbm, dst_hbm, zero_hbm, agg_hbm, aggS,
          mb0, mb1, mb2, mb3, db0, db1, db2, db3,
          ms0, ms1, ms2, ms3, ds0, ds1, ds2, ds3, ss0, ss1, ss2, ss3):
        cid = lax.axis_index("c")
        sid = lax.axis_index("s")
        ebase0 = cid * half + sid * per_tile
        mbs = (mb0, mb1, mb2, mb3)
        dbs = (db0, db1, db2, db3)
        msems = (ms0, ms1, ms2, ms3)
        dsems = (ds0, ds1, ds2, ds3)
        ssems = (ss0, ss1, ss2, ss3)

        pltpu.sync_copy(zero_hbm.at[pl.ds(sid * rows, rows)],
                        aggS.at[pl.ds(sid * rows, rows)])
        if tail:
            @pl.when(sid == 0)
            def _():
                pltpu.sync_copy(zero_hbm.at[pl.ds(NS * rows, tail)],
                                aggS.at[pl.ds(NS * rows, tail)])
        plsc.subcore_barrier()

        def issue_loads(ci, b):
            pltpu.async_copy(m_hbm.at[pl.ds(ebase0 + ci * CE, CE)],
                             mbs[b], msems[b])
            pltpu.async_copy(dst_hbm.at[pl.ds(ebase0 + ci * CE, CE)],
                             dbs[b], dsems[b])

        def group(g, _):
            for b4 in range(4):
                it = g * 4 + b4

                @pl.when(it < nch)
                def _():
                    b = b4
                    @pl.when(it >= 4)
                    def _():
                        pltpu.make_async_copy(
                            mbs[b], aggS.at[dbs[b]], ssems[b]).wait()
                    issue_loads(it, b)

                ci = it - 2
                @pl.when((ci >= 0) & (ci < nch))
                def _():
                    b = (b4 + 2) % 4
                    pltpu.make_async_copy(
                        m_hbm.at[pl.ds(ebase0 + ci * CE, CE)],
                        mbs[b], msems[b]).wait()
                    pltpu.make_async_copy(
                        dst_hbm.at[pl.ds(ebase0 + ci * CE, CE)],
                        dbs[b], dsems[b]).wait()
                    pltpu.async_copy(mbs[b], aggS.at[dbs[b]], ssems[b],
                                     add=True)
            return 0

        lax.fori_loop(0, ngrp, group, 0)
        for k4 in range(4):
            ci = nch - 4 + k4
            if ci >= 0:
                b = ci % 4
                pltpu.make_async_copy(mbs[b], aggS.at[dbs[b]],
                                      ssems[b]).wait()
        plsc.subcore_barrier()
        pltpu.sync_copy(aggS.at[pl.ds(sid * rows, rows)],
                        agg_hbm.at[pl.ds(cid * BN + sid * rows, rows)])
        if tail:
            @pl.when(sid == 0)
            def _():
                pltpu.sync_copy(aggS.at[pl.ds(NS * rows, tail)],
                                agg_hbm.at[pl.ds(cid * BN + NS * rows, tail)])

    return k


def _node_body(h2_ref, a0_ref, a1_ref, wa_ref, wb_ref, b1_ref, w2_ref, b2_ref,
               out_ref):
    agg = a0_ref[...] + a1_ref[...]
    u = (jnp.dot(h2_ref[...], wa_ref[...], preferred_element_type=jnp.float32)
         + jnp.dot(agg, wb_ref[...], preferred_element_type=jnp.float32)
         + b1_ref[...])
    u = u * jax.nn.sigmoid(u)
    out_ref[...] = (h2_ref[...]
                    + jnp.dot(u, w2_ref[...], preferred_element_type=jnp.float32)
                    + b2_ref[...])


def _node_slice(t, h2cat, part, wa, wb, b1, w2, b2, BN, C, BR):
    nblk = BN // BR
    return pl.pallas_call(
        _node_body,
        grid=(nblk,),
        in_specs=[
            pl.BlockSpec((BR, C), lambda i: (i, t)),
            pl.BlockSpec((BR, C), lambda i: (i, 0)),
            pl.BlockSpec((BR, C), lambda i: (nblk + i, 0)),
            pl.BlockSpec((C, C), lambda i: (0, 0)),
            pl.BlockSpec((C, C), lambda i: (0, 0)),
            pl.BlockSpec((1, C), lambda i: (0, 0)),
            pl.BlockSpec((C, C), lambda i: (0, 0)),
            pl.BlockSpec((1, C), lambda i: (0, 0)),
        ],
        out_specs=pl.BlockSpec((BR, C), lambda i: (i, 0)),
        out_shape=jax.ShapeDtypeStruct((BN, C), jnp.float32),
    )(h2cat, part, part, wa, wb, b1, w2, b2)


def kernel(h, x, vel_all, edge_index, tc_h_wr, tc_h_wi, tc_v_wr, tc_v_wi,
           W_e1, b_e1, W_e2, b_e2, W_g, b_g, W_n1, b_n1, W_n2, b_n2):
    T, BN, C = h.shape
    E = edge_index.shape[1]
    TE = T * E

    acat = _build_time_mats(tc_h_wr, tc_h_wi, T)
    a_v = _build_time_mats(tc_v_wr, tc_v_wi, T)
    bv = jnp.kron(a_v, jnp.eye(3, dtype=jnp.float32))
    w1a, w1b = W_e1[:C], W_e1[C:2 * C]
    w1c = W_e1[2 * C].reshape(1, C)
    src0 = edge_index[0].astype(jnp.int32)
    dst0 = edge_index[1].astype(jnp.int32)

    vflat = vel_all.reshape(BN, T * 3)
    h2cat, ptab, qtab, vnew = _prelude(h, acat, w1a, w1b, vflat, bv,
                                       T, BN, C, BR=2000)
    vel_out = vnew.reshape(BN, T, 3)

    XW = 16
    x16 = jnp.zeros((BN, XW), jnp.float32).at[:, :3].set(x)
    xs_g, xd_g = _gather_kernel(E, XW, CE=1000, tc_tiling=False)(
        x16, x16, src0, dst0)

    offs = jnp.repeat(jnp.arange(T, dtype=jnp.int32) * BN, E)
    src_all = jnp.tile(src0, T) + offs
    dst_all = jnp.tile(dst0, T) + offs
    g = _gather_add_kernel(TE, C, CE=200)(
        ptab.reshape(T * BN, C), qtab.reshape(T * BN, C), src_all, dst_all)

    bgr = jnp.full((1, C), b_g[0] / C, jnp.float32)
    zeros = jnp.zeros((BN, C), jnp.float32)
    scat = _scatter_slice_kernel(BN, E, C, CE=40)
    h_outs = []
    for t in range(T):
        m_t = _edge_mlp_slice(t, g, xs_g, xd_g, w1c, b_e1.reshape(1, C), W_e2,
                              b_e2.reshape(1, C), W_g.reshape(1, C), bgr,
                              E, C, XW, BE=4000)
        part_t = scat(m_t, dst0, zeros)
        h_outs.append(_node_slice(t, h2cat, part_t, W_n1[:C], W_n1[C:],
                                  b_n1.reshape(1, C), W_n2,
                                  b_n2.reshape(1, C), BN, C, BR=2000))
    return jnp.stack(h_outs, axis=0), vel_out


# speedup vs baseline: 1.0687x; 1.0062x over previous
import functools

import jax
import jax.numpy as jnp
import numpy as np
from jax import lax
from jax.experimental import pallas as pl
from jax.experimental.pallas import tpu as pltpu
from jax.experimental.pallas import tpu_sc as plsc

NC, NS, LANES = 2, 16, 16
NW = NC * NS


def _build_time_mats(wr, wi, T):
    tp = np.arange(T)[:, None].astype(np.float32)
    t = np.arange(T)[None, :].astype(np.float32)
    th = np.pi * (t - tp) / 2.0
    c = jnp.asarray(np.cos(th))
    s = jnp.asarray(np.sin(th))
    sign = jnp.asarray(((-1.0) ** (t + tp)).astype(np.float32))
    A = (wr[None, None, :, :, 0]
         + 2.0 * (c[:, :, None, None] * wr[None, None, :, :, 1]
                  - s[:, :, None, None] * wi[None, None, :, :, 1])
         + sign[:, :, None, None] * wr[None, None, :, :, 2]) / 4.0
    Ti, _, Cin, Cout = A.shape
    return jnp.transpose(A, (0, 2, 1, 3)).reshape(Ti * Cin, Ti * Cout)


def _prelude_body(h_ref, acat_ref, w1a_ref, w1b_ref, v_ref, bv_ref,
                  h2_ref, p_ref, q_ref, vn_ref):
    T = h_ref.shape[0]
    C = h_ref.shape[2]
    hcat = jnp.concatenate([h_ref[t] for t in range(T)], axis=-1)
    xh = jnp.dot(hcat, acat_ref[...], preferred_element_type=jnp.float32)
    h2 = hcat + jnp.where(xh > 0, xh, 0.2 * xh)
    h2_ref[...] = h2
    for t in range(T):
        blk = h2[:, t * C:(t + 1) * C]
        p_ref[t] = jnp.dot(blk, w1a_ref[...], preferred_element_type=jnp.float32)
        q_ref[t] = jnp.dot(blk, w1b_ref[...], preferred_element_type=jnp.float32)
    v = v_ref[...]
    vn_ref[...] = v + jnp.dot(v, bv_ref[...], preferred_element_type=jnp.float32)


def _prelude(h, acat, w1a, w1b, v, bv, T, BN, C, BR):
    nblk = BN // BR
    return pl.pallas_call(
        _prelude_body,
        grid=(nblk,),
        in_specs=[
            pl.BlockSpec((T, BR, C), lambda i: (0, i, 0)),
            pl.BlockSpec((T * C, T * C), lambda i: (0, 0)),
            pl.BlockSpec((C, C), lambda i: (0, 0)),
            pl.BlockSpec((C, C), lambda i: (0, 0)),
            pl.BlockSpec((BR, 3 * T), lambda i: (i, 0)),
            pl.BlockSpec((3 * T, 3 * T), lambda i: (0, 0)),
        ],
        out_specs=[
            pl.BlockSpec((BR, T * C), lambda i: (i, 0)),
            pl.BlockSpec((T, BR, C), lambda i: (0, i, 0)),
            pl.BlockSpec((T, BR, C), lambda i: (0, i, 0)),
            pl.BlockSpec((BR, 3 * T), lambda i: (i, 0)),
        ],
        out_shape=[
            jax.ShapeDtypeStruct((BN, T * C), jnp.float32),
            jax.ShapeDtypeStruct((T, BN, C), jnp.float32),
            jax.ShapeDtypeStruct((T, BN, C), jnp.float32),
            jax.ShapeDtypeStruct((BN, 3 * T), jnp.float32),
        ],
    )(h, acat, w1a, w1b, v, bv)


def _gather_kernel(TE, C, CE, tc_tiling=True, dtype=jnp.float32):
    per_w = TE // NW
    nch = per_w // CE

    @functools.partial(
        pl.kernel,
        out_type=[jax.ShapeDtypeStruct((TE, C), dtype),
                  jax.ShapeDtypeStruct((TE, C), dtype)],
        compiler_params=None if tc_tiling else pltpu.CompilerParams(
            use_tc_tiling_on_sc=False),
        mesh=plsc.VectorSubcoreMesh(core_axis_name="c", subcore_axis_name="s"),
        scratch_types=[
            pltpu.VMEM((CE,), jnp.int32),
            pltpu.VMEM((CE,), jnp.int32),
            pltpu.VMEM((CE, C), dtype),
            pltpu.VMEM((CE, C), dtype),
            pltpu.SemaphoreType.DMA,
            pltpu.SemaphoreType.DMA,
        ],
    )
    def k(ptab, qtab, src_hbm, dst_hbm, gp_hbm, gq_hbm,
          sidx, didx, gpb, gqb, sem1, sem2):
        wid = lax.axis_index("c") * NS + lax.axis_index("s")
        wbase = wid * per_w

        def body(ci, _):
            base = wbase + ci * CE
            pltpu.sync_copy(src_hbm.at[pl.ds(base, CE)], sidx)
            pltpu.sync_copy(dst_hbm.at[pl.ds(base, CE)], didx)
            cp1 = pltpu.async_copy(ptab.at[sidx], gpb, sem1)
            cp2 = pltpu.async_copy(qtab.at[didx], gqb, sem2)
            cp1.wait()
            cp2.wait()
            pltpu.sync_copy(gpb, gp_hbm.at[pl.ds(base, CE)])
            pltpu.sync_copy(gqb, gq_hbm.at[pl.ds(base, CE)])
            return 0

        lax.fori_loop(0, nch, body, 0)

    return k


def _gather_add_kernel(TE, C, CE, dtype=jnp.float32):
    per_w = TE // NW
    nch = per_w // CE
    niter = nch + 2
    ngrp = (niter + 2) // 3

    @functools.partial(
        pl.kernel,
        out_type=jax.ShapeDtypeStruct((TE, C), dtype),
        mesh=plsc.VectorSubcoreMesh(core_axis_name="c", subcore_axis_name="s"),
        scratch_types=[
            pltpu.VMEM((per_w,), jnp.int32),
            pltpu.VMEM((per_w,), jnp.int32),
            pltpu.VMEM((CE, C), dtype),
            pltpu.VMEM((CE, C), dtype),
            pltpu.VMEM((CE, C), dtype),
            pltpu.SemaphoreType.DMA,
            pltpu.SemaphoreType.DMA,
            pltpu.SemaphoreType.DMA,
            pltpu.SemaphoreType.DMA,
            pltpu.SemaphoreType.DMA,
            pltpu.SemaphoreType.DMA,
            pltpu.SemaphoreType.DMA,
            pltpu.SemaphoreType.DMA,
            pltpu.SemaphoreType.DMA,
        ],
    )
    def k(ptab, qtab, src_hbm, dst_hbm, g_hbm, sall, dall,
          gb0, gb1, gb2, qs0, qs1, qs2, ps0, ps1, ps2, ws0, ws1, ws2):
        wid = lax.axis_index("c") * NS + lax.axis_index("s")
        wbase = wid * per_w
        gbs = (gb0, gb1, gb2)
        qsems = (qs0, qs1, qs2)
        psems = (ps0, ps1, ps2)
        wsems = (ws0, ws1, ws2)
        pltpu.sync_copy(src_hbm.at[pl.ds(wbase, per_w)], sall)
        pltpu.sync_copy(dst_hbm.at[pl.ds(wbase, per_w)], dall)

        def group(g, _):
            for b3 in range(3):
                it = g * 3 + b3

                @pl.when(it < nch)
                def _():
                    b = b3
                    @pl.when(it >= 3)
                    def _():
                        pltpu.make_async_copy(
                            gbs[b], g_hbm.at[pl.ds(wbase + (it - 3) * CE, CE)],
                            wsems[b]).wait()
                    pltpu.async_copy(
                        qtab.at[dall.at[pl.ds(it * CE, CE)]], gbs[b], qsems[b])

                ci_p = it - 1
                @pl.when((ci_p >= 0) & (ci_p < nch))
                def _():
                    b = (b3 + 2) % 3
                    pltpu.make_async_copy(
                        qtab.at[dall.at[pl.ds(ci_p * CE, CE)]], gbs[b],
                        qsems[b]).wait()
                    pltpu.async_copy(
                        ptab.at[sall.at[pl.ds(ci_p * CE, CE)]], gbs[b],
                        psems[b], add=True)

                ci_w = it - 2
                @pl.when((ci_w >= 0) & (ci_w < nch))
                def _():
                    b = (b3 + 1) % 3
                    pltpu.make_async_copy(
                        ptab.at[sall.at[pl.ds(ci_w * CE, CE)]], gbs[b],
                        psems[b]).wait()
                    pltpu.async_copy(
                        gbs[b], g_hbm.at[pl.ds(wbase + ci_w * CE, CE)],
                        wsems[b])
            return 0

        lax.fori_loop(0, ngrp, group, 0)
        for k3 in range(3):
            ci = nch - 3 + k3
            if ci >= 0:
                b = ci % 3
                pltpu.make_async_copy(
                    gbs[b], g_hbm.at[pl.ds(wbase + ci * CE, CE)],
                    wsems[b]).wait()

    return k


def _edge_body(g_ref, xs_ref, xd_ref, w1c_ref, b1_ref, we2_ref, b2_ref,
               wg_ref, bg_ref, out_ref):
    rel = xs_ref[...] - xd_ref[...]
    mask3 = lax.broadcasted_iota(jnp.int32, (1, rel.shape[1]), 1) < 3
    d2 = jnp.sum(jnp.where(mask3, rel * rel, 0.0), axis=-1, keepdims=True)
    g = (g_ref[...] + d2 * w1c_ref[...] + b1_ref[...])
    m1 = g * jax.nn.sigmoid(g)
    m2 = jnp.dot(m1, we2_ref[...], preferred_element_type=jnp.float32) + b2_ref[...]
    m2 = m2 * jax.nn.sigmoid(m2)
    gate = jax.nn.sigmoid(
        jnp.sum(m2 * wg_ref[...] + bg_ref[...], axis=-1, keepdims=True))
    out_ref[...] = m2 * gate


def _edge_mlp_slice(t, g, xs, xd, w1c, b1, we2, b2, wgr, bgr, E, C, XW, BE):
    nblk = E // BE
    return pl.pallas_call(
        _edge_body,
        grid=(nblk,),
        in_specs=[
            pl.BlockSpec((BE, C), lambda i: (t * nblk + i, 0)),
            pl.BlockSpec((BE, XW), lambda i: (i, 0)),
            pl.BlockSpec((BE, XW), lambda i: (i, 0)),
            pl.BlockSpec((1, C), lambda i: (0, 0)),
            pl.BlockSpec((1, C), lambda i: (0, 0)),
            pl.BlockSpec((C, C), lambda i: (0, 0)),
            pl.BlockSpec((1, C), lambda i: (0, 0)),
            pl.BlockSpec((1, C), lambda i: (0, 0)),
            pl.BlockSpec((1, C), lambda i: (0, 0)),
        ],
        out_specs=pl.BlockSpec((BE, C), lambda i: (i, 0)),
        out_shape=jax.ShapeDtypeStruct((E, C), jnp.float32),
    )(g, xs, xd, w1c, b1, we2, b2, wgr, bgr)


def _scatter_slice_kernel(BN, E, C, CE):
    half = E // NC
    per_tile = half // NS
    nch = per_tile // CE
    niter = nch + 2
    ngrp = (niter + 3) // 4
    rows = (BN // NS) // 8 * 8
    tail = BN - rows * NS

    @functools.partial(
        pl.kernel,
        out_type=jax.ShapeDtypeStruct((NC * BN, C), jnp.float32),
        mesh=plsc.VectorSubcoreMesh(core_axis_name="c", subcore_axis_name="s"),
        scratch_types=[
            pltpu.VMEM_SHARED((BN, C), jnp.float32),
        ] + [pltpu.VMEM((CE, C), jnp.float32)] * 4
          + [pltpu.VMEM((CE,), jnp.int32)] * 4
          + [pltpu.SemaphoreType.DMA] * 12,
    )
    def k(m_hbm, dst_hbm, zero_hbm, agg_hbm, aggS,
          mb0, mb1, mb2, mb3, db0, db1, db2, db3,
          ms0, ms1, ms2, ms3, ds0, ds1, ds2, ds3, ss0, ss1, ss2, ss3):
        cid = lax.axis_index("c")
        sid = lax.axis_index("s")
        ebase0 = cid * half + sid * per_tile
        mbs = (mb0, mb1, mb2, mb3)
        dbs = (db0, db1, db2, db3)
        msems = (ms0, ms1, ms2, ms3)
        dsems = (ds0, ds1, ds2, ds3)
        ssems = (ss0, ss1, ss2, ss3)

        pltpu.sync_copy(zero_hbm.at[pl.ds(sid * rows, rows)],
                        aggS.at[pl.ds(sid * rows, rows)])
        if tail:
            @pl.when(sid == 0)
            def _():
                pltpu.sync_copy(zero_hbm.at[pl.ds(NS * rows, tail)],
                                aggS.at[pl.ds(NS * rows, tail)])
        plsc.subcore_barrier()

        def issue_loads(ci, b):
            pltpu.async_copy(m_hbm.at[pl.ds(ebase0 + ci * CE, CE)],
                             mbs[b], msems[b])
            pltpu.async_copy(dst_hbm.at[pl.ds(ebase0 + ci * CE, CE)],
                             dbs[b], dsems[b])

        def group(g, _):
            for b4 in range(4):
                it = g * 4 + b4

                @pl.when(it < nch)
                def _():
                    b = b4
                    @pl.when(it >= 4)
                    def _():
                        pltpu.make_async_copy(
                            mbs[b], aggS.at[dbs[b]], ssems[b]).wait()
                    issue_loads(it, b)

                ci = it - 2
                @pl.when((ci >= 0) & (ci < nch))
                def _():
                    b = (b4 + 2) % 4
                    pltpu.make_async_copy(
                        m_hbm.at[pl.ds(ebase0 + ci * CE, CE)],
                        mbs[b], msems[b]).wait()
                    pltpu.make_async_copy(
                        dst_hbm.at[pl.ds(ebase0 + ci * CE, CE)],
                        dbs[b], dsems[b]).wait()
                    pltpu.async_copy(mbs[b], aggS.at[dbs[b]], ssems[b],
                                     add=True)
            return 0

        lax.fori_loop(0, ngrp, group, 0)
        for k4 in range(4):
            ci = nch - 4 + k4
            if ci >= 0:
                b = ci % 4
                pltpu.make_async_copy(mbs[b], aggS.at[dbs[b]],
                                      ssems[b]).wait()
        plsc.subcore_barrier()
        pltpu.sync_copy(aggS.at[pl.ds(sid * rows, rows)],
                        agg_hbm.at[pl.ds(cid * BN + sid * rows, rows)])
        if tail:
            @pl.when(sid == 0)
            def _():
                pltpu.sync_copy(aggS.at[pl.ds(NS * rows, tail)],
                                agg_hbm.at[pl.ds(cid * BN + NS * rows, tail)])

    return k


def _node_body(h2_ref, a0_ref, a1_ref, wa_ref, wb_ref, b1_ref, w2_ref, b2_ref,
               out_ref):
    agg = a0_ref[...] + a1_ref[...]
    u = (jnp.dot(h2_ref[...], wa_ref[...], preferred_element_type=jnp.float32)
         + jnp.dot(agg, wb_ref[...], preferred_element_type=jnp.float32)
         + b1_ref[...])
    u = u * jax.nn.sigmoid(u)
    out_ref[...] = (h2_ref[...]
                    + jnp.dot(u, w2_ref[...], preferred_element_type=jnp.float32)
                    + b2_ref[...])


def _node_slice(t, h2cat, part, wa, wb, b1, w2, b2, BN, C, BR):
    nblk = BN // BR
    return pl.pallas_call(
        _node_body,
        grid=(nblk,),
        in_specs=[
            pl.BlockSpec((BR, C), lambda i: (i, t)),
            pl.BlockSpec((BR, C), lambda i: (i, 0)),
            pl.BlockSpec((BR, C), lambda i: (nblk + i, 0)),
            pl.BlockSpec((C, C), lambda i: (0, 0)),
            pl.BlockSpec((C, C), lambda i: (0, 0)),
            pl.BlockSpec((1, C), lambda i: (0, 0)),
            pl.BlockSpec((C, C), lambda i: (0, 0)),
            pl.BlockSpec((1, C), lambda i: (0, 0)),
        ],
        out_specs=pl.BlockSpec((BR, C), lambda i: (i, 0)),
        out_shape=jax.ShapeDtypeStruct((BN, C), jnp.float32),
    )(h2cat, part, part, wa, wb, b1, w2, b2)


def kernel(h, x, vel_all, edge_index, tc_h_wr, tc_h_wi, tc_v_wr, tc_v_wi,
           W_e1, b_e1, W_e2, b_e2, W_g, b_g, W_n1, b_n1, W_n2, b_n2):
    T, BN, C = h.shape
    E = edge_index.shape[1]
    TE = T * E

    acat = _build_time_mats(tc_h_wr, tc_h_wi, T)
    a_v = _build_time_mats(tc_v_wr, tc_v_wi, T)
    bv = jnp.kron(a_v, jnp.eye(3, dtype=jnp.float32))
    w1a, w1b = W_e1[:C], W_e1[C:2 * C]
    w1c = W_e1[2 * C].reshape(1, C)
    src0 = edge_index[0].astype(jnp.int32)
    dst0 = edge_index[1].astype(jnp.int32)

    vflat = vel_all.reshape(BN, T * 3)
    h2cat, ptab, qtab, vnew = _prelude(h, acat, w1a, w1b, vflat, bv,
                                       T, BN, C, BR=2000)
    vel_out = vnew.reshape(BN, T, 3)

    XW = 16
    x16 = jnp.zeros((BN, XW), jnp.float32).at[:, :3].set(x)
    xs_g, xd_g = _gather_kernel(E, XW, CE=1000, tc_tiling=False)(
        x16, x16, src0, dst0)

    offs = jnp.repeat(jnp.arange(T, dtype=jnp.int32) * BN, E)
    src_all = jnp.tile(src0, T) + offs
    dst_all = jnp.tile(dst0, T) + offs
    g = _gather_add_kernel(TE, C, CE=200)(
        ptab.reshape(T * BN, C), qtab.reshape(T * BN, C), src_all, dst_all)

    bgr = jnp.full((1, C), b_g[0] / C, jnp.float32)
    zeros = jnp.zeros((BN, C), jnp.float32)
    scat = _scatter_slice_kernel(BN, E, C, CE=40)
    h_outs = []
    for t in range(T):
        m_t = _edge_mlp_slice(t, g, xs_g, xd_g, w1c, b_e1.reshape(1, C), W_e2,
                              b_e2.reshape(1, C), W_g.reshape(1, C), bgr,
                              E, C, XW, BE=8000)
        part_t = scat(m_t, dst0, zeros)
        h_outs.append(_node_slice(t, h2cat, part_t, W_n1[:C], W_n1[C:],
                                  b_n1.reshape(1, C), W_n2,
                                  b_n2.reshape(1, C), BN, C, BR=2000))
    return jnp.stack(h_outs, axis=0), vel_out
